# baseline (speedup 1.0000x reference)
import functools

import jax
import jax.numpy as jnp
from jax.experimental import pallas as pl
from jax.experimental.pallas import tpu as pltpu

B, D, H, L, K = 4096, 768, 512, 64, 512
BLK = 2048


def _vqvae_kernel(x_ref, w1_ref, b1_ref, w2_ref, b2_ref, w3_ref, b3_ref,
                  cb_ref, dw1_ref, db1_ref, dw2_ref, db2_ref, dw3_ref, db3_ref,
                  xr_ref, idx_ref, loss_ref, tab_ref, acc_ref):
    f32 = jnp.float32
    nb = pl.num_programs(0)
    i = pl.program_id(0)

    @pl.when(i == 0)
    def _build_table():
        q = cb_ref[...]
        t = jax.nn.relu(jnp.dot(q, dw1_ref[...], preferred_element_type=f32) + db1_ref[...][None, :])
        t = jax.nn.relu(jnp.dot(t, dw2_ref[...], preferred_element_type=f32) + db2_ref[...][None, :])
        t = jnp.dot(t, dw3_ref[...], preferred_element_type=f32) + db3_ref[...][None, :]
        tab_ref[...] = t.astype(jnp.bfloat16)
        acc_ref[0] = 0.0

    x = x_ref[...]
    h = jax.nn.relu(jnp.dot(x, w1_ref[...], preferred_element_type=f32) + b1_ref[...][None, :])
    h = jax.nn.relu(jnp.dot(h, w2_ref[...], preferred_element_type=f32) + b2_ref[...][None, :])
    z = jnp.dot(h, w3_ref[...], preferred_element_type=f32) + b3_ref[...][None, :]

    cb = cb_ref[...]
    cnorm = jnp.sum(cb * cb, axis=1)[None, :]
    znorm = jnp.sum(z * z, axis=1, keepdims=True)
    scores = jnp.dot(z, cb.T, preferred_element_type=f32)
    d = znorm - 2.0 * scores + cnorm

    dmin = jnp.min(d, axis=1, keepdims=True)
    iota_k = jax.lax.broadcasted_iota(jnp.int32, d.shape, 1)
    idx = jnp.min(jnp.where(d == dmin, iota_k, K), axis=1)
    idx_ref[...] = idx

    acc_ref[0] += jnp.sum(dmin)

    @pl.when(i == nb - 1)
    def _finalize_loss():
        cl = acc_ref[0] / (B * L)
        loss_ref[...] = jnp.reshape(cl + 0.25 * cl, (1,))

    onehot = (iota_k == idx[:, None]).astype(jnp.bfloat16)
    xr_ref[...] = jnp.dot(onehot, tab_ref[...], preferred_element_type=f32)


@functools.partial(jax.jit, static_argnames=("interpret",))
def _run(x, enc_W1, enc_b1, enc_W2, enc_b2, enc_W3, enc_b3, codebook,
         dec_W1, dec_b1, dec_W2, dec_b2, dec_W3, dec_b3, interpret=False):
    nb = B // BLK
    full = lambda a: pl.BlockSpec(a.shape, lambda i: (0,) * a.ndim)

    grid_spec = pltpu.PrefetchScalarGridSpec(
        num_scalar_prefetch=0,
        grid=(nb,),
        in_specs=[
            pl.BlockSpec((BLK, D), lambda i: (i, 0)),
            full(enc_W1), full(enc_b1), full(enc_W2), full(enc_b2),
            full(enc_W3), full(enc_b3), full(codebook),
            full(dec_W1), full(dec_b1), full(dec_W2), full(dec_b2),
            full(dec_W3), full(dec_b3),
        ],
        out_specs=[
            pl.BlockSpec((BLK, D), lambda i: (i, 0)),
            pl.BlockSpec((BLK,), lambda i: (i,)),
            pl.BlockSpec((1,), lambda i: (0,)),
        ],
        scratch_shapes=[pltpu.VMEM((K, D), jnp.bfloat16),
                        pltpu.SMEM((1,), jnp.float32)],
    )
    out_shape = [
        jax.ShapeDtypeStruct((B, D), jnp.float32),
        jax.ShapeDtypeStruct((B,), jnp.int32),
        jax.ShapeDtypeStruct((1,), jnp.float32),
    ]
    xr, idx1d, loss1 = pl.pallas_call(
        _vqvae_kernel,
        grid_spec=grid_spec,
        out_shape=out_shape,
        compiler_params=pltpu.CompilerParams(
            dimension_semantics=("arbitrary",),
        ),
        interpret=interpret,
    )(x, enc_W1, enc_b1, enc_W2, enc_b2, enc_W3, enc_b3, codebook,
      dec_W1, dec_b1, dec_W2, dec_b2, dec_W3, dec_b3)
    return xr, loss1.reshape(()), idx1d


def kernel(x, enc_W1, enc_b1, enc_W2, enc_b2, enc_W3, enc_b3, codebook,
           dec_W1, dec_b1, dec_W2, dec_b2, dec_W3, dec_b3):
    return _run(x, enc_W1, enc_b1, enc_W2, enc_b2, enc_W3, enc_b3, codebook,
                dec_W1, dec_b1, dec_W2, dec_b2, dec_W3, dec_b3)


# speedup vs baseline: 1.1612x; 1.1612x over previous
import functools

import jax
import jax.numpy as jnp
from jax.experimental import pallas as pl
from jax.experimental.pallas import tpu as pltpu

B, D, H, L, K = 4096, 768, 512, 64, 512
BLK = 512
NT = B // BLK


def _vqvae_kernel(x_ref, w1_ref, b1_ref, w2_ref, b2_ref, w3_ref, b3_ref,
                  cb_ref, dw1_ref, db1_ref, dw2_ref, db2_ref, dw3_ref, db3_ref,
                  xr_ref, idx_ref, loss_ref, tab_ref, dbuf_ref, acc_ref):
    f32 = jnp.float32
    i = pl.program_id(0)

    @pl.when(i == 0)
    def _build_table():
        q = cb_ref[...]
        t = jax.nn.relu(jnp.dot(q, dw1_ref[...], preferred_element_type=f32) + db1_ref[...][None, :])
        t = jax.nn.relu(jnp.dot(t, dw2_ref[...], preferred_element_type=f32) + db2_ref[...][None, :])
        t = jnp.dot(t, dw3_ref[...], preferred_element_type=f32) + db3_ref[...][None, :]
        tab_ref[...] = t.astype(jnp.bfloat16)
        acc_ref[0] = 0.0

    @pl.when(i < NT)
    def _stage_a():
        x = x_ref[...]
        h = jax.nn.relu(jnp.dot(x, w1_ref[...], preferred_element_type=f32) + b1_ref[...][None, :])
        h = jax.nn.relu(jnp.dot(h, w2_ref[...], preferred_element_type=f32) + b2_ref[...][None, :])
        z = jnp.dot(h, w3_ref[...], preferred_element_type=f32) + b3_ref[...][None, :]
        cb = cb_ref[...]
        cnorm = jnp.sum(cb * cb, axis=1)[None, :]
        znorm = jnp.sum(z * z, axis=1, keepdims=True)
        scores = jnp.dot(z, cb.T, preferred_element_type=f32)
        dbuf_ref[i % 2] = znorm - 2.0 * scores + cnorm

    @pl.when(i > 0)
    def _stage_b():
        d = dbuf_ref[(i + 1) % 2]
        dmin = jnp.min(d, axis=1, keepdims=True)
        iota_k = jax.lax.broadcasted_iota(jnp.int32, d.shape, 1)
        idx = jnp.min(jnp.where(d == dmin, iota_k, K), axis=1)
        idx_ref[...] = idx
        acc_ref[0] += jnp.sum(dmin)
        onehot = (iota_k == idx[:, None]).astype(jnp.bfloat16)
        xr_ref[...] = jnp.dot(onehot, tab_ref[...], preferred_element_type=f32)

    @pl.when(i == NT)
    def _finalize_loss():
        cl = acc_ref[0] / (B * L)
        loss_ref[...] = jnp.reshape(cl + 0.25 * cl, (1,))


@functools.partial(jax.jit, static_argnames=("interpret",))
def _run(x, enc_W1, enc_b1, enc_W2, enc_b2, enc_W3, enc_b3, codebook,
         dec_W1, dec_b1, dec_W2, dec_b2, dec_W3, dec_b3, interpret=False):
    full = lambda a: pl.BlockSpec(a.shape, lambda i: (0,) * a.ndim)

    grid_spec = pltpu.PrefetchScalarGridSpec(
        num_scalar_prefetch=0,
        grid=(NT + 1,),
        in_specs=[
            pl.BlockSpec((BLK, D), lambda i: (jnp.minimum(i, NT - 1), 0)),
            full(enc_W1), full(enc_b1), full(enc_W2), full(enc_b2),
            full(enc_W3), full(enc_b3), full(codebook),
            full(dec_W1), full(dec_b1), full(dec_W2), full(dec_b2),
            full(dec_W3), full(dec_b3),
        ],
        out_specs=[
            pl.BlockSpec((BLK, D), lambda i: (jnp.maximum(i - 1, 0), 0)),
            pl.BlockSpec((BLK,), lambda i: (jnp.maximum(i - 1, 0),)),
            pl.BlockSpec((1,), lambda i: (0,)),
        ],
        scratch_shapes=[pltpu.VMEM((K, D), jnp.bfloat16),
                        pltpu.VMEM((2, BLK, K), jnp.float32),
                        pltpu.SMEM((1,), jnp.float32)],
    )
    out_shape = [
        jax.ShapeDtypeStruct((B, D), jnp.float32),
        jax.ShapeDtypeStruct((B,), jnp.int32),
        jax.ShapeDtypeStruct((1,), jnp.float32),
    ]
    xr, idx1d, loss1 = pl.pallas_call(
        _vqvae_kernel,
        grid_spec=grid_spec,
        out_shape=out_shape,
        compiler_params=pltpu.CompilerParams(
            dimension_semantics=("arbitrary",),
        ),
        interpret=interpret,
    )(x, enc_W1, enc_b1, enc_W2, enc_b2, enc_W3, enc_b3, codebook,
      dec_W1, dec_b1, dec_W2, dec_b2, dec_W3, dec_b3)
    return xr, loss1.reshape(()), idx1d


def kernel(x, enc_W1, enc_b1, enc_W2, enc_b2, enc_W3, enc_b3, codebook,
           dec_W1, dec_b1, dec_W2, dec_b2, dec_W3, dec_b3):
    return _run(x, enc_W1, enc_b1, enc_W2, enc_b2, enc_W3, enc_b3, codebook,
                dec_W1, dec_b1, dec_W2, dec_b2, dec_W3, dec_b3)


# speedup vs baseline: 1.2817x; 1.1038x over previous
import functools

import jax
import jax.numpy as jnp
from jax.experimental import pallas as pl
from jax.experimental.pallas import tpu as pltpu

B, D, H, L, K = 4096, 768, 512, 64, 512
BLK = 1024
NT = B // BLK


def _vqvae_kernel(x_ref, w1_ref, b1_ref, w2_ref, b2_ref, w3_ref, b3_ref,
                  cb_ref, dw1_ref, db1_ref, dw2_ref, db2_ref, dw3_ref, db3_ref,
                  xr_ref, idx_ref, loss_ref, tab_ref, dbuf_ref, acc_ref):
    f32 = jnp.float32
    i = pl.program_id(0)

    @pl.when(i == 0)
    def _build_table():
        q = cb_ref[...]
        t = jax.nn.relu(jnp.dot(q, dw1_ref[...], preferred_element_type=f32) + db1_ref[...][None, :])
        t = jax.nn.relu(jnp.dot(t, dw2_ref[...], preferred_element_type=f32) + db2_ref[...][None, :])
        t = jnp.dot(t, dw3_ref[...], preferred_element_type=f32) + db3_ref[...][None, :]
        tab_ref[...] = t.astype(jnp.bfloat16)
        acc_ref[0] = 0.0

    @pl.when(i < NT)
    def _stage_a():
        x = x_ref[...]
        h = jax.nn.relu(jnp.dot(x, w1_ref[...], preferred_element_type=f32) + b1_ref[...][None, :])
        h = jax.nn.relu(jnp.dot(h, w2_ref[...], preferred_element_type=f32) + b2_ref[...][None, :])
        z = jnp.dot(h, w3_ref[...], preferred_element_type=f32) + b3_ref[...][None, :]
        cb = cb_ref[...]
        cnorm = jnp.sum(cb * cb, axis=1)[None, :]
        znorm = jnp.sum(z * z, axis=1, keepdims=True)
        scores = jnp.dot(z, cb.T, preferred_element_type=f32)
        dbuf_ref[i % 2] = znorm - 2.0 * scores + cnorm

    @pl.when(i > 0)
    def _stage_b():
        d = dbuf_ref[(i + 1) % 2]
        dmin = jnp.min(d, axis=1, keepdims=True)
        iota_k = jax.lax.broadcasted_iota(jnp.int32, d.shape, 1)
        idx = jnp.min(jnp.where(d == dmin, iota_k, K), axis=1)
        idx_ref[...] = idx
        acc_ref[0] += jnp.sum(dmin)
        onehot = (iota_k == idx[:, None]).astype(jnp.bfloat16)
        xr_ref[...] = jnp.dot(onehot, tab_ref[...], preferred_element_type=f32)

    @pl.when(i == NT)
    def _finalize_loss():
        cl = acc_ref[0] / (B * L)
        loss_ref[...] = jnp.reshape(cl + 0.25 * cl, (1,))


@functools.partial(jax.jit, static_argnames=("interpret",))
def _run(x, enc_W1, enc_b1, enc_W2, enc_b2, enc_W3, enc_b3, codebook,
         dec_W1, dec_b1, dec_W2, dec_b2, dec_W3, dec_b3, interpret=False):
    full = lambda a: pl.BlockSpec(a.shape, lambda i: (0,) * a.ndim)

    grid_spec = pltpu.PrefetchScalarGridSpec(
        num_scalar_prefetch=0,
        grid=(NT + 1,),
        in_specs=[
            pl.BlockSpec((BLK, D), lambda i: (jnp.minimum(i, NT - 1), 0)),
            full(enc_W1), full(enc_b1), full(enc_W2), full(enc_b2),
            full(enc_W3), full(enc_b3), full(codebook),
            full(dec_W1), full(dec_b1), full(dec_W2), full(dec_b2),
            full(dec_W3), full(dec_b3),
        ],
        out_specs=[
            pl.BlockSpec((BLK, D), lambda i: (jnp.maximum(i - 1, 0), 0)),
            pl.BlockSpec((BLK,), lambda i: (jnp.maximum(i - 1, 0),)),
            pl.BlockSpec((1,), lambda i: (0,)),
        ],
        scratch_shapes=[pltpu.VMEM((K, D), jnp.bfloat16),
                        pltpu.VMEM((2, BLK, K), jnp.float32),
                        pltpu.SMEM((1,), jnp.float32)],
    )
    out_shape = [
        jax.ShapeDtypeStruct((B, D), jnp.float32),
        jax.ShapeDtypeStruct((B,), jnp.int32),
        jax.ShapeDtypeStruct((1,), jnp.float32),
    ]
    xr, idx1d, loss1 = pl.pallas_call(
        _vqvae_kernel,
        grid_spec=grid_spec,
        out_shape=out_shape,
        compiler_params=pltpu.CompilerParams(
            dimension_semantics=("arbitrary",),
        ),
        interpret=interpret,
    )(x, enc_W1, enc_b1, enc_W2, enc_b2, enc_W3, enc_b3, codebook,
      dec_W1, dec_b1, dec_W2, dec_b2, dec_W3, dec_b3)
    return xr, loss1.reshape(()), idx1d


def kernel(x, enc_W1, enc_b1, enc_W2, enc_b2, enc_W3, enc_b3, codebook,
           dec_W1, dec_b1, dec_W2, dec_b2, dec_W3, dec_b3):
    return _run(x, enc_W1, enc_b1, enc_W2, enc_b2, enc_W3, enc_b3, codebook,
                dec_W1, dec_b1, dec_W2, dec_b2, dec_W3, dec_b3)
